# Initial kernel scaffold; baseline (speedup 1.0000x reference)
import jax
import jax.numpy as jnp
from jax.experimental import pallas as pl


def kernel(logits_s, logits_w, y, T, Y_tilde, Y_hat):
    raise NotImplementedError("write your pallas kernel here")


# speedup vs baseline: 3.0641x; 3.0641x over previous
import functools

import jax
import jax.numpy as jnp
from jax import lax
from jax.experimental import pallas as pl
from jax.experimental.pallas import tpu as pltpu
from jax.experimental.pallas import tpu_sc as plsc

TEMPERATURE = 0.5
THRESHOLD = 0.95

_LN2 = 0.6931471805599453


def _vlog(x):
    bits = lax.bitcast_convert_type(x, jnp.int32)
    e = lax.shift_right_logical(bits, 23) - 127
    mb = lax.bitwise_or(lax.bitwise_and(bits, 0x7FFFFF), 0x3F800000)
    m = lax.bitcast_convert_type(mb, jnp.float32)
    t = (m - 1.0) / (m + 1.0)
    t2 = t * t
    p = t * (2.0 + t2 * (0.6666666666 + t2 * (0.4 + t2 * (0.2857142857 + t2 * 0.2222222222))))
    return e.astype(jnp.float32) * _LN2 + p


def _make_sc_call(B, C, N, NP):
    info = plsc.get_sparse_core_info()
    NW = info.num_cores * info.num_subcores
    rows_w = B // NW
    groups = rows_w // 16
    hgroups = NP // 16
    CW = C * 16

    mesh = plsc.VectorSubcoreMesh(core_axis_name="c", subcore_axis_name="s")

    @functools.partial(
        pl.kernel,
        mesh=mesh,
        compiler_params=pltpu.CompilerParams(needs_layout_passes=False),
        out_type=jax.ShapeDtypeStruct((NW, 16), jnp.float32),
        scratch_types=[
            pltpu.VMEM((rows_w * C,), jnp.float32),
            pltpu.VMEM((rows_w * C,), jnp.float32),
            pltpu.VMEM((rows_w,), jnp.int32),
            pltpu.VMEM((C * C,), jnp.float32),
            pltpu.VMEM((NP,), jnp.int32),
            pltpu.VMEM((NP,), jnp.int32),
            pltpu.VMEM(((C + 1) * 16,), jnp.float32),
            pltpu.VMEM((C * 16,), jnp.float32),
            pltpu.VMEM((16,), jnp.float32),
            pltpu.VMEM((16,), jnp.float32),
        ],
    )
    def sc_call(ls_hbm, lw_hbm, y_hbm, T_hbm, yt_hbm, yh_hbm, out_hbm,
                ls_v, lw_v, y_v, T_v, yt_v, yh_v, cnt_v, M_v, beta_v, ob_v):
        wid = lax.axis_index("s") * info.num_cores + lax.axis_index("c")
        base = wid * (rows_w * C)
        pltpu.sync_copy(ls_hbm.at[pl.ds(base, rows_w * C)], ls_v)
        pltpu.sync_copy(lw_hbm.at[pl.ds(base, rows_w * C)], lw_v)
        pltpu.sync_copy(y_hbm.at[pl.ds(wid * rows_w, rows_w)], y_v)
        pltpu.sync_copy(T_hbm, T_v)
        pltpu.sync_copy(yt_hbm, yt_v)
        pltpu.sync_copy(yh_hbm, yh_v)

        iota = lax.iota(jnp.int32, 16)
        zeros = jnp.zeros((16,), jnp.float32)
        ones = jnp.ones((16,), jnp.float32)

        for j in range(C + 1):
            cnt_v[pl.ds(j * 16, 16)] = zeros

        def hist_body(g, carry):
            ty = yt_v[pl.ds(g * 16, 16)]
            th = yh_v[pl.ds(g * 16, 16)]
            k = ty * 16 + th
            for l in range(16):
                cur = plsc.load_gather(cnt_v, [k])
                plsc.store_scatter(cnt_v, [k], cur + 1.0, mask=iota == l)
            return carry

        lax.fori_loop(0, hgroups, hist_body, 0)

        colsum = zeros
        dy = zeros
        rows = []
        for a in range(C):
            row = cnt_v[pl.ds(a * 16, 16)]
            rows.append(row)
            colsum = colsum + row
            dy = jnp.where(iota == a, jnp.sum(row) * (1.0 / N), dy)
        msum = zeros
        mrows = []
        for a in range(C):
            c10 = plsc.load_gather(cnt_v, [jnp.full((16,), a * 16 + C, jnp.int32)])
            mrow = rows[a] + c10 * dy
            mrows.append(mrow)
            msum = msum + mrow
        for a in range(C):
            M_v[pl.ds(a * 16, 16)] = mrows[a] / msum
        colsum = colsum
        bmax = jnp.max(colsum)
        beta_v[...] = colsum / (2.0 * bmax - colsum)

        iotaC = iota * C

        def grp(g, acc):
            rowoff = g * (16 * C) + iotaC
            yv = y_v[pl.ds(g * 16, 16)]
            yvC = yv * C
            yv16 = yv * 16
            lw = [plsc.load_gather(lw_v, [rowoff + c]) for c in range(C)]
            m = lw[0]
            for c in range(1, C):
                m = jnp.maximum(m, lw[c])
            best = None
            for c in range(C):
                e = jnp.exp((lw[c] - m) * (1.0 / TEMPERATURE))
                t = plsc.load_gather(T_v, [yv + c * C])
                mm = plsc.load_gather(M_v, [yv16 + c])
                w = e * t / mm
                if c == 0:
                    best, barg, ssum = w, jnp.zeros((16,), jnp.int32), w
                else:
                    gt = w > best
                    barg = jnp.where(gt, c, barg)
                    best = jnp.where(gt, w, best)
                    ssum = ssum + w
            maxp = best / ssum
            bt = plsc.load_gather(beta_v, [barg])
            msk = maxp > THRESHOLD * bt
            ls = [plsc.load_gather(ls_v, [rowoff + c]) for c in range(C)]
            m2 = ls[0]
            for c in range(1, C):
                m2 = jnp.maximum(m2, ls[c])
            s2 = jnp.exp(ls[0] - m2)
            for c in range(1, C):
                s2 = s2 + jnp.exp(ls[c] - m2)
            lst = plsc.load_gather(ls_v, [rowoff + barg])
            ce = m2 + _vlog(s2) - lst
            return acc + jnp.where(msk, ce, 0.0)

        acc = lax.fori_loop(0, groups, grp, zeros)
        total = jnp.sum(acc)
        ob_v[...] = jnp.where(iota == 0, total, 0.0)
        pltpu.sync_copy(ob_v, out_hbm.at[wid])

    return sc_call


def kernel(logits_s, logits_w, y, T, Y_tilde, Y_hat):
    B, C = logits_s.shape
    N = Y_tilde.shape[0]
    NP = ((N + 15) // 16) * 16
    yt_p = jnp.concatenate(
        [Y_tilde.astype(jnp.int32), jnp.full((NP - N,), C, jnp.int32)])
    yh_p = jnp.concatenate(
        [Y_hat.astype(jnp.int32), jnp.zeros((NP - N,), jnp.int32)])
    sc_call = _make_sc_call(B, C, N, NP)
    partials = sc_call(
        logits_s.reshape(-1), logits_w.reshape(-1), y.astype(jnp.int32),
        T.reshape(-1), yt_p, yh_p)
    return jnp.sum(partials) / B


# speedup vs baseline: 3.2418x; 1.0580x over previous
import functools

import jax
import jax.numpy as jnp
from jax import lax
from jax.experimental import pallas as pl
from jax.experimental.pallas import tpu as pltpu
from jax.experimental.pallas import tpu_sc as plsc

TEMPERATURE = 0.5
THRESHOLD = 0.95

_LN2 = 0.6931471805599453


def _vlog(x):
    bits = lax.bitcast_convert_type(x, jnp.int32)
    e = lax.shift_right_logical(bits, 23) - 127
    mb = lax.bitwise_or(lax.bitwise_and(bits, 0x7FFFFF), 0x3F800000)
    m = lax.bitcast_convert_type(mb, jnp.float32)
    t = (m - 1.0) / (m + 1.0)
    t2 = t * t
    p = t * (2.0 + t2 * (0.6666666666 + t2 * (0.4 + t2 * (0.2857142857 + t2 * 0.2222222222))))
    return e.astype(jnp.float32) * _LN2 + p


def _make_sc_call(B, C, N, NP):
    info = plsc.get_sparse_core_info()
    NW = info.num_cores * info.num_subcores
    rows_w = B // NW
    groups = rows_w // 16
    hgroups = NP // 16
    CW = C * 16

    mesh = plsc.VectorSubcoreMesh(core_axis_name="c", subcore_axis_name="s")

    @functools.partial(
        pl.kernel,
        mesh=mesh,
        compiler_params=pltpu.CompilerParams(needs_layout_passes=False),
        out_type=jax.ShapeDtypeStruct((NW, 16), jnp.float32),
        scratch_types=[
            pltpu.VMEM((rows_w * C,), jnp.float32),
            pltpu.VMEM((rows_w * C,), jnp.float32),
            pltpu.VMEM((rows_w,), jnp.int32),
            pltpu.VMEM((C * 16,), jnp.float32),
            pltpu.VMEM((NP,), jnp.int32),
            pltpu.VMEM((NP,), jnp.int32),
            pltpu.VMEM(((C + 1) * 16 * 16,), jnp.float32),
            pltpu.VMEM((256,), jnp.float32),
            pltpu.VMEM((C * 16,), jnp.float32),
            pltpu.VMEM((16,), jnp.float32),
            pltpu.VMEM((16,), jnp.float32),
        ],
    )
    def sc_call(ls_hbm, lw_hbm, y_hbm, T_hbm, yt_hbm, yh_hbm, out_hbm,
                ls_v, lw_v, y_v, T_v, yt_v, yh_v, cnt_v, M_v, F_v, beta_v, ob_v):
        wid = lax.axis_index("s") * info.num_cores + lax.axis_index("c")
        base = wid * (rows_w * C)
        pltpu.sync_copy(ls_hbm.at[pl.ds(base, rows_w * C)], ls_v)
        pltpu.sync_copy(lw_hbm.at[pl.ds(base, rows_w * C)], lw_v)
        pltpu.sync_copy(y_hbm.at[pl.ds(wid * rows_w, rows_w)], y_v)
        pltpu.sync_copy(T_hbm, T_v)
        pltpu.sync_copy(yt_hbm, yt_v)
        pltpu.sync_copy(yh_hbm, yh_v)

        iota = lax.iota(jnp.int32, 16)
        zeros = jnp.zeros((16,), jnp.float32)
        ones = jnp.ones((16,), jnp.float32)

        def zero_body(j, carry):
            cnt_v[pl.ds(j * 16, 16)] = zeros
            return carry

        lax.fori_loop(0, (C + 1) * 16, zero_body, 0)

        def hist_body(g, carry):
            ty = yt_v[pl.ds(g * 16, 16)]
            th = yh_v[pl.ds(g * 16, 16)]
            k = (ty * 16 + th) * 16 + iota
            cur = plsc.load_gather(cnt_v, [k])
            plsc.store_scatter(cnt_v, [k], cur + 1.0)
            return carry

        lax.fori_loop(0, hgroups, hist_body, 0)

        colsum = zeros
        dy = zeros
        rows = []
        for a in range(C):
            row = zeros
            for l in range(16):
                row = row + plsc.load_gather(cnt_v, [a * 256 + iota * 16 + l])
            rows.append(row)
            colsum = colsum + row
            dy = jnp.where(iota == a, jnp.sum(row) * (1.0 / N), dy)
        msum = zeros
        for a in range(C):
            c10 = jnp.sum(jnp.where(iota == C, rows[a], 0.0))
            mrow = rows[a] + c10 * dy
            M_v[pl.ds(a * 16, 16)] = mrow
            msum = msum + mrow
        for c in range(C):
            trow = T_v[pl.ds(c * 16, 16)]
            mcol = plsc.load_gather(M_v, [iota * 16 + c])
            msum_c = jnp.sum(jnp.where(iota == c, msum, 0.0))
            F_v[pl.ds(c * 16, 16)] = trow * msum_c / mcol
        bmax = jnp.max(colsum)
        beta_v[...] = colsum / (2.0 * bmax - colsum)

        iotaC = iota * C

        def grp(g, acc):
            rowoff = g * (16 * C) + iotaC
            yv = y_v[pl.ds(g * 16, 16)]
            lw = [plsc.load_gather(lw_v, [rowoff + c]) for c in range(C)]
            m = lw[0]
            for c in range(1, C):
                m = jnp.maximum(m, lw[c])
            best = None
            for c in range(C):
                e = jnp.exp((lw[c] - m) * (1.0 / TEMPERATURE))
                f = plsc.load_gather(F_v, [yv + c * 16])
                w = e * f
                if c == 0:
                    best, barg, ssum = w, jnp.zeros((16,), jnp.int32), w
                else:
                    gt = w > best
                    barg = jnp.where(gt, c, barg)
                    best = jnp.where(gt, w, best)
                    ssum = ssum + w
            maxp = best / ssum
            bt = plsc.load_gather(beta_v, [barg])
            msk = maxp > THRESHOLD * bt
            ls = [plsc.load_gather(ls_v, [rowoff + c]) for c in range(C)]
            m2 = ls[0]
            for c in range(1, C):
                m2 = jnp.maximum(m2, ls[c])
            s2 = jnp.exp(ls[0] - m2)
            for c in range(1, C):
                s2 = s2 + jnp.exp(ls[c] - m2)
            lst = plsc.load_gather(ls_v, [rowoff + barg])
            ce = m2 + _vlog(s2) - lst
            return acc + jnp.where(msk, ce, 0.0)

        acc = lax.fori_loop(0, groups, grp, zeros)
        total = jnp.sum(acc)
        ob_v[...] = jnp.where(iota == 0, total, 0.0)
        pltpu.sync_copy(ob_v, out_hbm.at[wid])

    return sc_call


def kernel(logits_s, logits_w, y, T, Y_tilde, Y_hat):
    B, C = logits_s.shape
    N = Y_tilde.shape[0]
    NP = ((N + 15) // 16) * 16
    yt_p = jnp.concatenate(
        [Y_tilde.astype(jnp.int32), jnp.full((NP - N,), C, jnp.int32)])
    yh_p = jnp.concatenate(
        [Y_hat.astype(jnp.int32), jnp.zeros((NP - N,), jnp.int32)])
    T16 = jnp.pad(T, ((0, 0), (0, 16 - C))).reshape(-1)
    sc_call = _make_sc_call(B, C, N, NP)
    partials = sc_call(
        logits_s.reshape(-1), logits_w.reshape(-1), y.astype(jnp.int32),
        T16, yt_p, yh_p)
    return jnp.sum(partials) / B


# speedup vs baseline: 4.4660x; 1.3776x over previous
import functools

import jax
import jax.numpy as jnp
from jax import lax
from jax.experimental import pallas as pl
from jax.experimental.pallas import tpu as pltpu
from jax.experimental.pallas import tpu_sc as plsc

TEMPERATURE = 0.5
THRESHOLD = 0.95

_LN2 = 0.6931471805599453


def _vlog(x):
    bits = lax.bitcast_convert_type(x, jnp.int32)
    e = lax.shift_right_logical(bits, 23) - 127
    mb = lax.bitwise_or(lax.bitwise_and(bits, 0x7FFFFF), 0x3F800000)
    m = lax.bitcast_convert_type(mb, jnp.float32)
    t = (m - 1.0) / (m + 1.0)
    t2 = t * t
    p = t * (2.0 + t2 * (0.6666666666 + t2 * (0.4 + t2 * (0.2857142857 + t2 * 0.2222222222))))
    return e.astype(jnp.float32) * _LN2 + p


def _make_sc_call(B, C, N, NP):
    info = plsc.get_sparse_core_info()
    NW = info.num_cores * info.num_subcores
    rows_w = B // NW
    RQ = 128
    quarters = rows_w // RQ
    hgroups = NP // 16

    mesh = plsc.VectorSubcoreMesh(core_axis_name="c", subcore_axis_name="s")

    @functools.partial(
        pl.kernel,
        mesh=mesh,
        compiler_params=pltpu.CompilerParams(needs_layout_passes=False),
        out_type=jax.ShapeDtypeStruct((NW, 16), jnp.float32),
        scratch_types=[
            pltpu.VMEM((RQ, C), jnp.float32),
            pltpu.VMEM((RQ, C), jnp.float32),
            pltpu.VMEM((RQ, C), jnp.float32),
            pltpu.VMEM((RQ, C), jnp.float32),
            pltpu.VMEM((rows_w,), jnp.int32),
            pltpu.VMEM((C, C), jnp.float32),
            pltpu.VMEM((NP,), jnp.int32),
            pltpu.VMEM((NP,), jnp.int32),
            pltpu.VMEM(((C + 1) * 16 * 16,), jnp.float32),
            pltpu.VMEM((256,), jnp.float32),
            pltpu.VMEM((C * 16,), jnp.float32),
            pltpu.VMEM((16,), jnp.float32),
            pltpu.VMEM((16,), jnp.float32),
            pltpu.SemaphoreType.DMA,
            pltpu.SemaphoreType.DMA,
            pltpu.SemaphoreType.DMA,
            pltpu.SemaphoreType.DMA,
            pltpu.SemaphoreType.DMA,
            pltpu.SemaphoreType.DMA,
            pltpu.SemaphoreType.DMA,
            pltpu.SemaphoreType.DMA,
        ],
    )
    def sc_call(ls_hbm, lw_hbm, y_hbm, T_hbm, yt_hbm, yh_hbm, out_hbm,
                ls0_v, ls1_v, lw0_v, lw1_v, y_v, T_v, yt_v, yh_v,
                cnt_v, M_v, F_v, beta_v, ob_v,
                s_yt, s_yh, s_T, s_y, s_lw0, s_lw1, s_ls0, s_ls1):
        wid = lax.axis_index("s") * info.num_cores + lax.axis_index("c")
        row0 = wid * rows_w
        lw_bufs, ls_bufs = [lw0_v, lw1_v], [ls0_v, ls1_v]
        lw_sems, ls_sems = [s_lw0, s_lw1], [s_ls0, s_ls1]
        cp_yt = pltpu.async_copy(yt_hbm, yt_v.at[pl.ds(0, N)], s_yt)
        cp_yh = pltpu.async_copy(yh_hbm, yh_v.at[pl.ds(0, N)], s_yh)
        cp_T = pltpu.async_copy(T_hbm, T_v, s_T)
        cp_y = pltpu.async_copy(y_hbm.at[pl.ds(row0, rows_w)], y_v, s_y)

        def fetch(q):
            b = q % 2
            lw_cp = pltpu.async_copy(
                lw_hbm.at[pl.ds(row0 + q * RQ, RQ), :], lw_bufs[b], lw_sems[b])
            ls_cp = pltpu.async_copy(
                ls_hbm.at[pl.ds(row0 + q * RQ, RQ), :], ls_bufs[b], ls_sems[b])
            return lw_cp, ls_cp

        cps = {0: fetch(0), 1: fetch(1)}

        iota = lax.iota(jnp.int32, 16)
        zeros = jnp.zeros((16,), jnp.float32)

        def zero_body(j, carry):
            cnt_v[pl.ds(j * 16, 16)] = zeros
            return carry

        lax.fori_loop(0, (C + 1) * 16, zero_body, 0)

        cp_yt.wait()
        cp_yh.wait()
        if N % 16:
            tail = NP - 16
            nvalid = N - tail
            tv = yt_v[pl.ds(tail, 16)]
            yt_v[pl.ds(tail, 16)] = jnp.where(iota < nvalid, tv, C)
            hv = yh_v[pl.ds(tail, 16)]
            yh_v[pl.ds(tail, 16)] = jnp.where(iota < nvalid, hv, 0)

        def hist_body(g, carry):
            ty = yt_v[pl.ds(g * 16, 16)]
            th = yh_v[pl.ds(g * 16, 16)]
            k = (ty * 16 + th) * 16 + iota
            cur = plsc.load_gather(cnt_v, [k])
            plsc.store_scatter(cnt_v, [k], cur + 1.0)
            return carry

        lax.fori_loop(0, hgroups, hist_body, 0)

        colsum = zeros
        dy = zeros
        rows = []
        for a in range(C):
            row = zeros
            for l in range(16):
                row = row + plsc.load_gather(cnt_v, [a * 256 + iota * 16 + l])
            rows.append(row)
            colsum = colsum + row
            dy = jnp.where(iota == a, jnp.sum(row) * (1.0 / N), dy)
        msum = zeros
        for a in range(C):
            c10 = jnp.sum(jnp.where(iota == C, rows[a], 0.0))
            mrow = rows[a] + c10 * dy
            M_v[pl.ds(a * 16, 16)] = mrow
            msum = msum + mrow
        cp_T.wait()
        lane_valid = iota < C
        cfull = jnp.zeros((16,), jnp.int32)
        for c in range(C):
            trow = plsc.load_gather(T_v, [cfull + c, iota], mask=lane_valid)
            mcol = plsc.load_gather(M_v, [iota * 16 + c])
            msum_c = jnp.sum(jnp.where(iota == c, msum, 0.0))
            F_v[pl.ds(c * 16, 16)] = trow * msum_c / mcol
        bmax = jnp.max(colsum)
        beta_v[...] = colsum / (2.0 * bmax - colsum)

        cp_y.wait()
        acc = zeros
        for q in range(quarters):
            lw_v, ls_v = lw_bufs[q % 2], ls_bufs[q % 2]
            lw_cp, ls_cp = cps.pop(q)
            lw_cp.wait()
            ls_cp.wait()

            def grp(g, acc, q=q, lw_v=lw_v, ls_v=ls_v):
                rvec = g * 16 + iota
                yv = y_v[pl.ds(q * RQ + g * 16, 16)]
                lw = [plsc.load_gather(lw_v, [rvec, cfull + c]) for c in range(C)]
                m = lw[0]
                for c in range(1, C):
                    m = jnp.maximum(m, lw[c])
                best = None
                for c in range(C):
                    e = jnp.exp((lw[c] - m) * (1.0 / TEMPERATURE))
                    f = plsc.load_gather(F_v, [yv + c * 16])
                    w = e * f
                    if c == 0:
                        best, barg, ssum = w, jnp.zeros((16,), jnp.int32), w
                    else:
                        gt = w > best
                        barg = jnp.where(gt, c, barg)
                        best = jnp.where(gt, w, best)
                        ssum = ssum + w
                maxp = best / ssum
                bt = plsc.load_gather(beta_v, [barg])
                msk = maxp > THRESHOLD * bt
                ls = [plsc.load_gather(ls_v, [rvec, cfull + c]) for c in range(C)]
                m2 = ls[0]
                for c in range(1, C):
                    m2 = jnp.maximum(m2, ls[c])
                s2 = jnp.exp(ls[0] - m2)
                for c in range(1, C):
                    s2 = s2 + jnp.exp(ls[c] - m2)
                lst = plsc.load_gather(ls_v, [rvec, barg])
                ce = m2 + _vlog(s2) - lst
                return acc + jnp.where(msk, ce, 0.0)

            acc = lax.fori_loop(0, RQ // 16, grp, acc)
            if q + 2 < quarters:
                cps[q + 2] = fetch(q + 2)
        total = jnp.sum(acc)
        ob_v[...] = jnp.where(iota == 0, total, 0.0)
        pltpu.sync_copy(ob_v, out_hbm.at[wid])

    return sc_call


def kernel(logits_s, logits_w, y, T, Y_tilde, Y_hat):
    B, C = logits_s.shape
    N = Y_tilde.shape[0]
    NP = ((N + 15) // 16) * 16
    sc_call = _make_sc_call(B, C, N, NP)
    partials = sc_call(logits_s, logits_w, y.astype(jnp.int32), T,
                       Y_tilde.astype(jnp.int32), Y_hat.astype(jnp.int32))
    return jnp.sum(partials) / B


# speedup vs baseline: 6.8414x; 1.5319x over previous
import functools

import jax
import jax.numpy as jnp
from jax import lax
from jax.experimental import pallas as pl
from jax.experimental.pallas import tpu as pltpu
from jax.experimental.pallas import tpu_sc as plsc

TEMPERATURE = 0.5
THRESHOLD = 0.95

_LN2 = 0.6931471805599453


def _vlog(x):
    bits = lax.bitcast_convert_type(x, jnp.int32)
    e = lax.shift_right_logical(bits, 23) - 127
    mb = lax.bitwise_or(lax.bitwise_and(bits, 0x7FFFFF), 0x3F800000)
    m = lax.bitcast_convert_type(mb, jnp.float32)
    t = (m - 1.0) / (m + 1.0)
    t2 = t * t
    p = t * (2.0 + t2 * (0.6666666666 + t2 * (0.4 + t2 * (0.2857142857 + t2 * 0.2222222222))))
    return e.astype(jnp.float32) * _LN2 + p


def _make_sc_call(B, C, N, NP):
    info = plsc.get_sparse_core_info()
    NW = info.num_cores * info.num_subcores
    rows_w = B // NW
    hgroups = NP // 16

    mesh = plsc.VectorSubcoreMesh(core_axis_name="c", subcore_axis_name="s")

    @functools.partial(
        pl.kernel,
        mesh=mesh,
        compiler_params=pltpu.CompilerParams(needs_layout_passes=False),
        out_type=jax.ShapeDtypeStruct((NW, 16), jnp.float32),
        scratch_types=[
            pltpu.VMEM((C, rows_w), jnp.float32),
            pltpu.VMEM((C, rows_w), jnp.float32),
            pltpu.VMEM((rows_w,), jnp.int32),
            pltpu.VMEM((C, C), jnp.float32),
            pltpu.VMEM((NP,), jnp.int32),
            pltpu.VMEM((NP,), jnp.int32),
            pltpu.VMEM(((C + 1) * 16 * 16,), jnp.float32),
            pltpu.VMEM((256,), jnp.float32),
            pltpu.VMEM((C * 16,), jnp.float32),
            pltpu.VMEM((16,), jnp.float32),
            pltpu.VMEM((16,), jnp.float32),
            pltpu.SemaphoreType.DMA,
            pltpu.SemaphoreType.DMA,
            pltpu.SemaphoreType.DMA,
            pltpu.SemaphoreType.DMA,
            pltpu.SemaphoreType.DMA,
            pltpu.SemaphoreType.DMA,
        ],
    )
    def sc_call(ls_hbm, lw_hbm, y_hbm, T_hbm, yt_hbm, yh_hbm, out_hbm,
                ls_v, lw_v, y_v, T_v, yt_v, yh_v,
                cnt_v, M_v, F_v, beta_v, ob_v,
                s_yt, s_yh, s_T, s_y, s_lw, s_ls):
        wid = lax.axis_index("s") * info.num_cores + lax.axis_index("c")
        row0 = wid * rows_w
        cp_yt = pltpu.async_copy(yt_hbm, yt_v.at[pl.ds(0, N)], s_yt)
        cp_yh = pltpu.async_copy(yh_hbm, yh_v.at[pl.ds(0, N)], s_yh)
        cp_T = pltpu.async_copy(T_hbm, T_v, s_T)
        cp_y = pltpu.async_copy(y_hbm.at[pl.ds(row0, rows_w)], y_v, s_y)
        cp_lw = pltpu.async_copy(lw_hbm.at[:, pl.ds(row0, rows_w)], lw_v, s_lw)
        cp_ls = pltpu.async_copy(ls_hbm.at[:, pl.ds(row0, rows_w)], ls_v, s_ls)

        iota = lax.iota(jnp.int32, 16)
        zeros = jnp.zeros((16,), jnp.float32)

        def zero_body(j, carry):
            cnt_v[pl.ds(j * 16, 16)] = zeros
            return carry

        lax.fori_loop(0, (C + 1) * 16, zero_body, 0)

        cp_yt.wait()
        cp_yh.wait()
        if N % 16:
            tail = NP - 16
            nvalid = N - tail
            tv = yt_v[pl.ds(tail, 16)]
            yt_v[pl.ds(tail, 16)] = jnp.where(iota < nvalid, tv, C)
            hv = yh_v[pl.ds(tail, 16)]
            yh_v[pl.ds(tail, 16)] = jnp.where(iota < nvalid, hv, 0)

        def hist_body(g, carry):
            ty = yt_v[pl.ds(g * 16, 16)]
            th = yh_v[pl.ds(g * 16, 16)]
            k = (ty * 16 + th) * 16 + iota
            cur = plsc.load_gather(cnt_v, [k])
            plsc.store_scatter(cnt_v, [k], cur + 1.0)
            return carry

        lax.fori_loop(0, hgroups, hist_body, 0)

        colsum = zeros
        dy = zeros
        rows = []
        for a in range(C):
            row = zeros
            for l in range(16):
                row = row + plsc.load_gather(cnt_v, [a * 256 + iota * 16 + l])
            rows.append(row)
            colsum = colsum + row
            dy = jnp.where(iota == a, jnp.sum(row) * (1.0 / N), dy)
        msum = zeros
        for a in range(C):
            c10 = jnp.sum(jnp.where(iota == C, rows[a], 0.0))
            mrow = rows[a] + c10 * dy
            M_v[pl.ds(a * 16, 16)] = mrow
            msum = msum + mrow
        cp_T.wait()
        lane_valid = iota < C
        cfull = jnp.zeros((16,), jnp.int32)
        for c in range(C):
            trow = plsc.load_gather(T_v, [cfull + c, iota], mask=lane_valid)
            mcol = plsc.load_gather(M_v, [iota * 16 + c])
            msum_c = jnp.sum(jnp.where(iota == c, msum, 0.0))
            F_v[pl.ds(c * 16, 16)] = trow * msum_c / mcol
        bmax = jnp.max(colsum)
        beta_v[...] = colsum / (2.0 * bmax - colsum)

        cp_y.wait()
        cp_lw.wait()
        cp_ls.wait()

        def grp(g, acc):
            rvec = g * 16 + iota
            yv = y_v[pl.ds(g * 16, 16)]
            lw = [lw_v[c, pl.ds(g * 16, 16)] for c in range(C)]
            m = lw[0]
            for c in range(1, C):
                m = jnp.maximum(m, lw[c])
            best = None
            for c in range(C):
                e = jnp.exp((lw[c] - m) * (1.0 / TEMPERATURE))
                f = plsc.load_gather(F_v, [yv + c * 16])
                w = e * f
                if c == 0:
                    best, barg, ssum = w, jnp.zeros((16,), jnp.int32), w
                else:
                    gt = w > best
                    barg = jnp.where(gt, c, barg)
                    best = jnp.where(gt, w, best)
                    ssum = ssum + w
            maxp = best / ssum
            bt = plsc.load_gather(beta_v, [barg])
            msk = maxp > THRESHOLD * bt
            ls = [ls_v[c, pl.ds(g * 16, 16)] for c in range(C)]
            m2 = ls[0]
            for c in range(1, C):
                m2 = jnp.maximum(m2, ls[c])
            s2 = jnp.exp(ls[0] - m2)
            for c in range(1, C):
                s2 = s2 + jnp.exp(ls[c] - m2)
            lst = plsc.load_gather(ls_v, [barg, rvec])
            ce = m2 + _vlog(s2) - lst
            return acc + jnp.where(msk, ce, 0.0)

        acc = lax.fori_loop(0, rows_w // 16, grp, zeros)
        total = jnp.sum(acc)
        ob_v[...] = jnp.where(iota == 0, total, 0.0)
        pltpu.sync_copy(ob_v, out_hbm.at[wid])

    return sc_call


def kernel(logits_s, logits_w, y, T, Y_tilde, Y_hat):
    B, C = logits_s.shape
    N = Y_tilde.shape[0]
    NP = ((N + 15) // 16) * 16
    sc_call = _make_sc_call(B, C, N, NP)
    partials = sc_call(logits_s.T, logits_w.T, y.astype(jnp.int32), T,
                       Y_tilde.astype(jnp.int32), Y_hat.astype(jnp.int32))
    return jnp.sum(partials) / B


# speedup vs baseline: 6.9619x; 1.0176x over previous
import functools

import jax
import jax.numpy as jnp
from jax import lax
from jax.experimental import pallas as pl
from jax.experimental.pallas import tpu as pltpu
from jax.experimental.pallas import tpu_sc as plsc

TEMPERATURE = 0.5
THRESHOLD = 0.95

_LN2 = 0.6931471805599453


def _vlog(x):
    bits = lax.bitcast_convert_type(x, jnp.int32)
    e = lax.shift_right_logical(bits, 23) - 127
    mb = lax.bitwise_or(lax.bitwise_and(bits, 0x7FFFFF), 0x3F800000)
    m = lax.bitcast_convert_type(mb, jnp.float32)
    t = (m - 1.0) / (m + 1.0)
    t2 = t * t
    p = t * (2.0 + t2 * (0.6666666666 + t2 * (0.4 + t2 * (0.2857142857 + t2 * 0.2222222222))))
    return e.astype(jnp.float32) * _LN2 + p


def _make_sc_call(B, C, N, NP):
    info = plsc.get_sparse_core_info()
    NW = info.num_cores * info.num_subcores
    rows_w = B // NW
    hgroups = NP // 16

    mesh = plsc.VectorSubcoreMesh(core_axis_name="c", subcore_axis_name="s")

    @functools.partial(
        pl.kernel,
        mesh=mesh,
        compiler_params=pltpu.CompilerParams(needs_layout_passes=False),
        out_type=jax.ShapeDtypeStruct((NW, 16), jnp.float32),
        scratch_types=[
            pltpu.VMEM((C, rows_w), jnp.float32),
            pltpu.VMEM((C, rows_w), jnp.float32),
            pltpu.VMEM((rows_w,), jnp.int32),
            pltpu.VMEM((C, C), jnp.float32),
            pltpu.VMEM((NP,), jnp.int32),
            pltpu.VMEM((NP,), jnp.int32),
            pltpu.VMEM(((C + 1) * 16 * 16,), jnp.float32),
            pltpu.VMEM((256,), jnp.float32),
            pltpu.VMEM((C * 16,), jnp.float32),
            pltpu.VMEM((16,), jnp.float32),
            pltpu.VMEM((16,), jnp.float32),
            pltpu.SemaphoreType.DMA,
            pltpu.SemaphoreType.DMA,
            pltpu.SemaphoreType.DMA,
            pltpu.SemaphoreType.DMA,
            pltpu.SemaphoreType.DMA,
            pltpu.SemaphoreType.DMA,
        ],
    )
    def sc_call(ls_hbm, lw_hbm, y_hbm, T_hbm, yt_hbm, yh_hbm, out_hbm,
                ls_v, lw_v, y_v, T_v, yt_v, yh_v,
                cnt_v, M_v, F_v, beta_v, ob_v,
                s_yt, s_yh, s_T, s_y, s_lw, s_ls):
        wid = lax.axis_index("s") * info.num_cores + lax.axis_index("c")
        row0 = wid * rows_w
        cp_yt = pltpu.async_copy(yt_hbm, yt_v.at[pl.ds(0, N)], s_yt)
        cp_yh = pltpu.async_copy(yh_hbm, yh_v.at[pl.ds(0, N)], s_yh)
        cp_T = pltpu.async_copy(T_hbm, T_v, s_T)
        cp_y = pltpu.async_copy(y_hbm.at[pl.ds(row0, rows_w)], y_v, s_y)
        cp_lw = pltpu.async_copy(lw_hbm.at[:, pl.ds(row0, rows_w)], lw_v, s_lw)
        cp_ls = pltpu.async_copy(ls_hbm.at[:, pl.ds(row0, rows_w)], ls_v, s_ls)

        iota = lax.iota(jnp.int32, 16)
        zeros = jnp.zeros((16,), jnp.float32)

        def zero_body(j, carry):
            cnt_v[pl.ds(j * 16, 16)] = zeros
            return carry

        lax.fori_loop(0, (C + 1) * 16, zero_body, 0)

        cp_yt.wait()
        cp_yh.wait()
        if N % 16:
            tail = NP - 16
            nvalid = N - tail
            tv = yt_v[pl.ds(tail, 16)]
            yt_v[pl.ds(tail, 16)] = jnp.where(iota < nvalid, tv, C)
            hv = yh_v[pl.ds(tail, 16)]
            yh_v[pl.ds(tail, 16)] = jnp.where(iota < nvalid, hv, 0)

        def hist_body(g, carry):
            ty = yt_v[pl.ds(g * 16, 16)]
            th = yh_v[pl.ds(g * 16, 16)]
            k = (ty * 16 + th) * 16 + iota
            cur = plsc.load_gather(cnt_v, [k])
            plsc.store_scatter(cnt_v, [k], cur + 1.0)
            return carry

        lax.fori_loop(0, hgroups, hist_body, 0)

        iota16 = iota * 16

        def comb(a, carry):
            colsum, dy = carry
            row = zeros
            for l in range(16):
                row = row + plsc.load_gather(cnt_v, [a * 256 + iota16 + l])
            M_v[pl.ds(a * 16, 16)] = row
            colsum = colsum + row
            dy = jnp.where(iota == a, jnp.sum(row) * (1.0 / N), dy)
            return colsum, dy

        colsum, dy = lax.fori_loop(0, C, comb, (zeros, zeros))

        def mpass(a, msum):
            row = M_v[pl.ds(a * 16, 16)]
            c10 = jnp.sum(jnp.where(iota == C, row, 0.0))
            mrow = row + c10 * dy
            M_v[pl.ds(a * 16, 16)] = mrow
            return msum + mrow

        msum = lax.fori_loop(0, C, mpass, zeros)

        cp_T.wait()
        lane_valid = iota < C
        cfull = jnp.zeros((16,), jnp.int32)

        def fpass(c, carry):
            trow = plsc.load_gather(T_v, [cfull + c, iota], mask=lane_valid)
            mcol = plsc.load_gather(M_v, [iota16 + c])
            msum_c = jnp.sum(jnp.where(iota == c, msum, 0.0))
            F_v[pl.ds(c * 16, 16)] = trow * msum_c / mcol
            return carry

        lax.fori_loop(0, C, fpass, 0)
        bmax = jnp.max(colsum)
        beta_v[...] = colsum / (2.0 * bmax - colsum)

        cp_y.wait()
        cp_lw.wait()
        cp_ls.wait()

        def grp(g, acc):
            rvec = g * 16 + iota
            yv = y_v[pl.ds(g * 16, 16)]
            lw = [lw_v[c, pl.ds(g * 16, 16)] for c in range(C)]
            m = lw[0]
            for c in range(1, C):
                m = jnp.maximum(m, lw[c])
            best = None
            for c in range(C):
                e = jnp.exp((lw[c] - m) * (1.0 / TEMPERATURE))
                f = plsc.load_gather(F_v, [yv + c * 16])
                w = e * f
                if c == 0:
                    best, barg, ssum = w, jnp.zeros((16,), jnp.int32), w
                else:
                    gt = w > best
                    barg = jnp.where(gt, c, barg)
                    best = jnp.where(gt, w, best)
                    ssum = ssum + w
            maxp = best / ssum
            bt = plsc.load_gather(beta_v, [barg])
            msk = maxp > THRESHOLD * bt
            ls = [ls_v[c, pl.ds(g * 16, 16)] for c in range(C)]
            m2 = ls[0]
            for c in range(1, C):
                m2 = jnp.maximum(m2, ls[c])
            s2 = jnp.exp(ls[0] - m2)
            for c in range(1, C):
                s2 = s2 + jnp.exp(ls[c] - m2)
            lst = plsc.load_gather(ls_v, [barg, rvec])
            ce = m2 + _vlog(s2) - lst
            return acc + jnp.where(msk, ce, 0.0)

        acc = lax.fori_loop(0, rows_w // 16, grp, zeros)
        total = jnp.sum(acc)
        ob_v[...] = jnp.where(iota == 0, total, 0.0)
        pltpu.sync_copy(ob_v, out_hbm.at[wid])

    return sc_call


def kernel(logits_s, logits_w, y, T, Y_tilde, Y_hat):
    B, C = logits_s.shape
    N = Y_tilde.shape[0]
    NP = ((N + 15) // 16) * 16
    sc_call = _make_sc_call(B, C, N, NP)
    partials = sc_call(logits_s.T, logits_w.T, y.astype(jnp.int32), T,
                       Y_tilde.astype(jnp.int32), Y_hat.astype(jnp.int32))
    return jnp.sum(partials) / B


# speedup vs baseline: 7.1939x; 1.0333x over previous
import functools

import jax
import jax.numpy as jnp
from jax import lax
from jax.experimental import pallas as pl
from jax.experimental.pallas import tpu as pltpu
from jax.experimental.pallas import tpu_sc as plsc

TEMPERATURE = 0.5
THRESHOLD = 0.95

_LN2 = 0.6931471805599453


def _vlog(x):
    bits = lax.bitcast_convert_type(x, jnp.int32)
    e = lax.shift_right_logical(bits, 23) - 127
    mb = lax.bitwise_or(lax.bitwise_and(bits, 0x7FFFFF), 0x3F800000)
    m = lax.bitcast_convert_type(mb, jnp.float32)
    t = (m - 1.0) / (m + 1.0)
    t2 = t * t
    p = t * (2.0 + t2 * (0.6666666666 + t2 * (0.4 + t2 * (0.2857142857 + t2 * 0.2222222222))))
    return e.astype(jnp.float32) * _LN2 + p


def _make_sc_call(B, C, N, NP):
    info = plsc.get_sparse_core_info()
    NW = info.num_cores * info.num_subcores
    rows_w = B // NW
    hgroups = NP // 16

    mesh = plsc.VectorSubcoreMesh(core_axis_name="c", subcore_axis_name="s")

    @functools.partial(
        pl.kernel,
        mesh=mesh,
        compiler_params=pltpu.CompilerParams(needs_layout_passes=False),
        out_type=jax.ShapeDtypeStruct((NW, 16), jnp.float32),
        scratch_types=[
            pltpu.VMEM((C, rows_w), jnp.float32),
            pltpu.VMEM((C, rows_w), jnp.float32),
            pltpu.VMEM((rows_w,), jnp.int32),
            pltpu.VMEM((C, C), jnp.float32),
            pltpu.VMEM((NP,), jnp.int32),
            pltpu.VMEM((NP,), jnp.int32),
            pltpu.VMEM(((C + 1) * 16,), jnp.float32),
            pltpu.VMEM((256,), jnp.float32),
            pltpu.VMEM((C * 16,), jnp.float32),
            pltpu.VMEM((16,), jnp.float32),
            pltpu.VMEM((16,), jnp.float32),
            pltpu.SemaphoreType.DMA,
            pltpu.SemaphoreType.DMA,
            pltpu.SemaphoreType.DMA,
            pltpu.SemaphoreType.DMA,
            pltpu.SemaphoreType.DMA,
            pltpu.SemaphoreType.DMA,
        ],
    )
    def sc_call(ls_hbm, lw_hbm, y_hbm, T_hbm, yt_hbm, yh_hbm, out_hbm,
                ls_v, lw_v, y_v, T_v, yt_v, yh_v,
                cnt_v, M_v, F_v, beta_v, ob_v,
                s_yt, s_yh, s_T, s_y, s_lw, s_ls):
        wid = lax.axis_index("s") * info.num_cores + lax.axis_index("c")
        row0 = wid * rows_w
        cp_yt = pltpu.async_copy(yt_hbm, yt_v.at[pl.ds(0, N)], s_yt)
        cp_yh = pltpu.async_copy(yh_hbm, yh_v.at[pl.ds(0, N)], s_yh)
        cp_T = pltpu.async_copy(T_hbm, T_v, s_T)
        cp_y = pltpu.async_copy(y_hbm.at[pl.ds(row0, rows_w)], y_v, s_y)
        cp_lw = pltpu.async_copy(lw_hbm.at[:, pl.ds(row0, rows_w)], lw_v, s_lw)
        cp_ls = pltpu.async_copy(ls_hbm.at[:, pl.ds(row0, rows_w)], ls_v, s_ls)

        iota = lax.iota(jnp.int32, 16)
        zeros = jnp.zeros((16,), jnp.float32)

        def zero_body(j, carry):
            cnt_v[pl.ds(j * 16, 16)] = zeros
            return carry

        lax.fori_loop(0, C + 1, zero_body, 0)

        cp_yt.wait()
        cp_yh.wait()
        if N % 16:
            tail = NP - 16
            nvalid = N - tail
            tv = yt_v[pl.ds(tail, 16)]
            yt_v[pl.ds(tail, 16)] = jnp.where(iota < nvalid, tv, C)
            hv = yh_v[pl.ds(tail, 16)]
            yh_v[pl.ds(tail, 16)] = jnp.where(iota < nvalid, hv, 0)

        ones = jnp.ones((16,), jnp.float32)

        def hist_body(g, carry):
            ty = yt_v[pl.ds(g * 16, 16)]
            th = yh_v[pl.ds(g * 16, 16)]
            plsc.addupdate_scatter(cnt_v, [ty * 16 + th], ones)
            return carry

        lax.fori_loop(0, hgroups, hist_body, 0)

        iota16 = iota * 16

        def comb(a, carry):
            colsum, dy = carry
            row = cnt_v[pl.ds(a * 16, 16)]
            colsum = colsum + row
            dy = jnp.where(iota == a, jnp.sum(row) * (1.0 / N), dy)
            return colsum, dy

        colsum, dy = lax.fori_loop(0, C, comb, (zeros, zeros))

        def mpass(a, msum):
            row = cnt_v[pl.ds(a * 16, 16)]
            c10 = jnp.sum(jnp.where(iota == C, row, 0.0))
            mrow = row + c10 * dy
            M_v[pl.ds(a * 16, 16)] = mrow
            return msum + mrow

        msum = lax.fori_loop(0, C, mpass, zeros)

        cp_T.wait()
        lane_valid = iota < C
        cfull = jnp.zeros((16,), jnp.int32)

        def fpass(c, carry):
            trow = plsc.load_gather(T_v, [cfull + c, iota], mask=lane_valid)
            mcol = plsc.load_gather(M_v, [iota16 + c])
            msum_c = jnp.sum(jnp.where(iota == c, msum, 0.0))
            F_v[pl.ds(c * 16, 16)] = trow * msum_c / mcol
            return carry

        lax.fori_loop(0, C, fpass, 0)
        bmax = jnp.max(colsum)
        beta_v[...] = colsum / (2.0 * bmax - colsum)

        cp_y.wait()
        cp_lw.wait()
        cp_ls.wait()

        def grp(g, acc):
            rvec = g * 16 + iota
            yv = y_v[pl.ds(g * 16, 16)]
            lw = [lw_v[c, pl.ds(g * 16, 16)] for c in range(C)]
            m = lw[0]
            for c in range(1, C):
                m = jnp.maximum(m, lw[c])
            best = None
            for c in range(C):
                e = jnp.exp((lw[c] - m) * (1.0 / TEMPERATURE))
                f = plsc.load_gather(F_v, [yv + c * 16])
                w = e * f
                if c == 0:
                    best, barg, ssum = w, jnp.zeros((16,), jnp.int32), w
                else:
                    gt = w > best
                    barg = jnp.where(gt, c, barg)
                    best = jnp.where(gt, w, best)
                    ssum = ssum + w
            maxp = best / ssum
            bt = plsc.load_gather(beta_v, [barg])
            msk = maxp > THRESHOLD * bt
            ls = [ls_v[c, pl.ds(g * 16, 16)] for c in range(C)]
            m2 = ls[0]
            for c in range(1, C):
                m2 = jnp.maximum(m2, ls[c])
            s2 = jnp.exp(ls[0] - m2)
            for c in range(1, C):
                s2 = s2 + jnp.exp(ls[c] - m2)
            lst = plsc.load_gather(ls_v, [barg, rvec])
            ce = m2 + _vlog(s2) - lst
            return acc + jnp.where(msk, ce, 0.0)

        acc = lax.fori_loop(0, rows_w // 16, grp, zeros)
        total = jnp.sum(acc)
        ob_v[...] = jnp.where(iota == 0, total, 0.0)
        pltpu.sync_copy(ob_v, out_hbm.at[wid])

    return sc_call


def kernel(logits_s, logits_w, y, T, Y_tilde, Y_hat):
    B, C = logits_s.shape
    N = Y_tilde.shape[0]
    NP = ((N + 15) // 16) * 16
    sc_call = _make_sc_call(B, C, N, NP)
    partials = sc_call(logits_s.T, logits_w.T, y.astype(jnp.int32), T,
                       Y_tilde.astype(jnp.int32), Y_hat.astype(jnp.int32))
    return jnp.sum(partials) / B
